# baseline (speedup 1.0000x reference)
import jax
import jax.numpy as jnp
from jax.experimental import pallas as pl
from jax.experimental.pallas import tpu as pltpu

_N = 10000
_NFEAT = 128
_NHID = 64
_NCLASS = 64

_BI = 200
_NROW = _N // _BI
_NBUF = 4


def _gcn_kernel(adj_hbm, x_ref, w1_ref, wh1_ref, wh2_ref, w2_ref, b1_ref,
                b2_ref, out_ref, bufs, sems, s1_scr, p_scr, s2_scr, s2b_scr):
    p = pl.program_id(0)
    i = pl.program_id(1)
    g = p * _NROW + i

    def dma_h(slot, blk, h):
        row = jnp.where(blk >= _NROW, blk - _NROW, blk) * _BI
        off, sz = (0, 96) if h == 0 else (96, 104)
        return pltpu.make_async_copy(
            adj_hbm.at[pl.ds(row + off, sz), :],
            bufs.at[slot, pl.ds(off, sz), :],
            sems.at[slot, h])

    class _Pair:
        def __init__(self, slot, blk):
            self.slot, self.blk = slot, blk

        def start(self):
            dma_h(self.slot, self.blk, 0).start()
            dma_h(self.slot, self.blk, 1).start()

        def wait(self):
            dma_h(self.slot, self.blk, 0).wait()
            dma_h(self.slot, self.blk, 1).wait()

    def dma(slot, blk):
        return _Pair(slot, blk)

    @pl.when(g == 0)
    def _prime():
        for s in range(_NBUF - 1):
            dma(s, s).start()

    slot = jax.lax.rem(g, _NBUF)
    nxt = g + _NBUF - 1

    @pl.when(nxt < 2 * _NROW)
    def _issue_ahead():
        dma(jax.lax.rem(nxt, _NBUF), nxt).start()

    @pl.when((p == 0) & (i == 0))
    def _prologue():
        xx = x_ref[...]
        s1_scr[...] = jnp.dot(xx, w1_ref[...] + wh1_ref[...],
                              preferred_element_type=jnp.float32
                              ).astype(jnp.bfloat16)
        p_scr[...] = jnp.dot(xx, wh2_ref[...],
                             preferred_element_type=jnp.float32)

    dma(slot, g).wait()
    adj_blk = bufs[slot]

    rows = pl.ds(pl.multiple_of(i * _BI, 8), _BI)

    @pl.when(p == 0)
    def _pass1():
        t = jnp.dot(adj_blk.astype(jnp.bfloat16), s1_scr[...],
                    preferred_element_type=jnp.float32)
        x1 = jnp.maximum(t + b1_ref[...], 0.0)
        s2_blk = (jnp.dot(x1, w2_ref[...], preferred_element_type=jnp.float32)
                  + p_scr[rows, :])
        s2_scr[rows, :] = s2_blk
        out_ref[...] = s2_blk

    @pl.when((p == 1) & (i == 0))
    def _cast_s2():
        s2b_scr[...] = s2_scr[...].astype(jnp.bfloat16)

    @pl.when(p == 1)
    def _pass2():
        out_ref[...] = (jnp.dot(adj_blk.astype(jnp.bfloat16), s2b_scr[...],
                                preferred_element_type=jnp.float32)
                        + b2_ref[...])


def kernel(adj, x, W1, Wh1, b1, W2, Wh2, b2):
    return pl.pallas_call(
        _gcn_kernel,
        grid=(2, _NROW),
        in_specs=[
            pl.BlockSpec(memory_space=pl.ANY),
            pl.BlockSpec((_N, _NFEAT), lambda p, i: (0, 0)),
            pl.BlockSpec((_NFEAT, _NHID), lambda p, i: (0, 0)),
            pl.BlockSpec((_NFEAT, _NHID), lambda p, i: (0, 0)),
            pl.BlockSpec((_NFEAT, _NCLASS), lambda p, i: (0, 0)),
            pl.BlockSpec((_NHID, _NCLASS), lambda p, i: (0, 0)),
            pl.BlockSpec((1, _NHID), lambda p, i: (0, 0)),
            pl.BlockSpec((1, _NCLASS), lambda p, i: (0, 0)),
        ],
        out_specs=pl.BlockSpec((_BI, _NCLASS), lambda p, i: (p * i, 0)),
        out_shape=jax.ShapeDtypeStruct((_N, _NCLASS), jnp.float32),
        scratch_shapes=[
            pltpu.VMEM((_NBUF, _BI, _N), jnp.float32),
            pltpu.SemaphoreType.DMA((_NBUF, 2)),
            pltpu.VMEM((_N, _NHID), jnp.bfloat16),
            pltpu.VMEM((_N, _NCLASS), jnp.float32),
            pltpu.VMEM((_N, _NCLASS), jnp.float32),
            pltpu.VMEM((_N, _NCLASS), jnp.bfloat16),
        ],
        compiler_params=pltpu.CompilerParams(
            dimension_semantics=("arbitrary", "arbitrary")),
    )(adj, x, W1, Wh1, Wh2, W2, b1.reshape(1, _NHID), b2.reshape(1, _NCLASS))


# speedup vs baseline: 1.0120x; 1.0120x over previous
import jax
import jax.numpy as jnp
from jax.experimental import pallas as pl
from jax.experimental.pallas import tpu as pltpu

_N = 10000
_NFEAT = 128
_NHID = 64
_NCLASS = 64

_BI = 400


def _gcn_kernel(adj_ref, x_ref, w1_ref, wh1_ref, wh2_ref, w2_ref, b1_ref,
                b2_ref, out_ref, s1_scr, p_scr, s2_scr):
    p = pl.program_id(0)
    i = pl.program_id(1)

    @pl.when((p == 0) & (i == 0))
    def _prologue():
        xx = x_ref[...]
        s1_scr[...] = jnp.dot(xx, w1_ref[...] + wh1_ref[...],
                              preferred_element_type=jnp.float32)
        p_scr[...] = jnp.dot(xx, wh2_ref[...],
                             preferred_element_type=jnp.float32)

    rows = pl.ds(pl.multiple_of(i * _BI, 8), _BI)

    @pl.when(p == 0)
    def _pass1():
        t = jnp.dot(adj_ref[...], s1_scr[...],
                    preferred_element_type=jnp.float32)
        x1 = jnp.maximum(t + b1_ref[...], 0.0)
        s2_blk = (jnp.dot(x1, w2_ref[...], preferred_element_type=jnp.float32)
                  + p_scr[rows, :])
        s2_scr[rows, :] = s2_blk
        out_ref[...] = s2_blk

    @pl.when(p == 1)
    def _pass2():
        out_ref[...] = (jnp.dot(adj_ref[...], s2_scr[...],
                                preferred_element_type=jnp.float32)
                        + b2_ref[...])


def kernel(adj, x, W1, Wh1, b1, W2, Wh2, b2):
    return pl.pallas_call(
        _gcn_kernel,
        grid=(2, _N // _BI),
        in_specs=[
            pl.BlockSpec((_BI, _N), lambda p, i: (i, 0)),
            pl.BlockSpec((_N, _NFEAT), lambda p, i: (0, 0)),
            pl.BlockSpec((_NFEAT, _NHID), lambda p, i: (0, 0)),
            pl.BlockSpec((_NFEAT, _NHID), lambda p, i: (0, 0)),
            pl.BlockSpec((_NFEAT, _NCLASS), lambda p, i: (0, 0)),
            pl.BlockSpec((_NHID, _NCLASS), lambda p, i: (0, 0)),
            pl.BlockSpec((1, _NHID), lambda p, i: (0, 0)),
            pl.BlockSpec((1, _NCLASS), lambda p, i: (0, 0)),
        ],
        out_specs=pl.BlockSpec((_BI, _NCLASS), lambda p, i: (p * i, 0)),
        out_shape=jax.ShapeDtypeStruct((_N, _NCLASS), jnp.float32),
        scratch_shapes=[
            pltpu.VMEM((_N, _NHID), jnp.float32),
            pltpu.VMEM((_N, _NCLASS), jnp.float32),
            pltpu.VMEM((_N, _NCLASS), jnp.float32),
        ],
        compiler_params=pltpu.CompilerParams(
            dimension_semantics=("arbitrary", "arbitrary")),
    )(adj, x, W1, Wh1, Wh2, W2, b1.reshape(1, _NHID), b2.reshape(1, _NCLASS))
